# baseline (speedup 1.0000x reference)
import functools

import jax
import jax.numpy as jnp
from jax import lax
from jax.experimental import pallas as pl
from jax.experimental.pallas import tpu as pltpu
from jax.experimental.pallas import tpu_sc as plsc

B = 16384
D = 1536
V = 32
L = 16
NC = 2
NS = 16
NW = NC * NS
B_PER_W = B // NW
CHUNK = 32
NCHUNK = B_PER_W // CHUNK


def _gather_body(rep_hbm, idx_hbm, out_hbm, idx_v, bufs, gsem, ssem):
    sid = lax.axis_index("s")
    wid = sid * NC + lax.axis_index("c")
    base = wid * B_PER_W

    pltpu.sync_copy(idx_hbm.at[pl.ds(base, B_PER_W)], idx_v)
    rebase = jnp.broadcast_to(wid * V, (L,)).astype(jnp.int32)
    for k in range(B_PER_W // L):
        idx_v[pl.ds(k * L, L)] = idx_v[pl.ds(k * L, L)] + rebase

    def idx_slice(g):
        return idx_v.at[pl.ds(g * CHUNK, CHUNK)]

    def out_slice(g):
        return out_hbm.at[pl.ds(base + g * CHUNK, CHUNK)]

    pltpu.async_copy(rep_hbm.at[idx_slice(0)], bufs.at[0], gsem)

    for g in range(NCHUNK):
        cur = g % 2
        pltpu.make_async_copy(rep_hbm.at[idx_slice(g)], bufs.at[cur], gsem).wait()
        if g >= 1:
            pltpu.make_async_copy(bufs.at[(g - 1) % 2], out_slice(g - 1), ssem).wait()
        if g + 1 < NCHUNK:
            pltpu.async_copy(rep_hbm.at[idx_slice(g + 1)], bufs.at[(g + 1) % 2], gsem)
        pltpu.async_copy(bufs.at[cur], out_slice(g), ssem)

    pltpu.make_async_copy(bufs.at[(NCHUNK - 1) % 2], out_slice(NCHUNK - 1), ssem).wait()


def kernel(b, cat_ids):
    cat_ids = cat_ids.astype(jnp.int32)
    rep = jnp.tile(b, (NW, 1))
    mesh = plsc.VectorSubcoreMesh(core_axis_name="c", subcore_axis_name="s")
    run = functools.partial(
        pl.kernel,
        mesh=mesh,
        compiler_params=pltpu.CompilerParams(needs_layout_passes=False),
        out_type=jax.ShapeDtypeStruct((B, D), jnp.float32),
        scratch_types=[
            pltpu.VMEM((B_PER_W,), jnp.int32),
            pltpu.VMEM((2, CHUNK, D), jnp.float32),
            pltpu.SemaphoreType.DMA,
            pltpu.SemaphoreType.DMA,
        ],
    )(_gather_body)
    return run(rep, cat_ids)


# speedup vs baseline: 1.0019x; 1.0019x over previous
import functools

import jax
import jax.numpy as jnp
from jax import lax
from jax.experimental import pallas as pl
from jax.experimental.pallas import tpu as pltpu
from jax.experimental.pallas import tpu_sc as plsc

B = 16384
D = 1536
V = 32
L = 16
NC = 2
NS = 16
NW = NC * NS
B_PER_W = B // NW
CHUNK = 16
NCHUNK = B_PER_W // CHUNK


def _gather_body(rep_hbm, idx_hbm, out_hbm, idx_v, bufs, gsem, ssem):
    sid = lax.axis_index("s")
    wid = sid * NC + lax.axis_index("c")
    base = wid * B_PER_W

    pltpu.sync_copy(idx_hbm.at[pl.ds(base, B_PER_W)], idx_v)
    for k in range(B_PER_W // L):
        rebase = jnp.broadcast_to((wid + (k % 2) * NW) * V, (L,)).astype(jnp.int32)
        idx_v[pl.ds(k * L, L)] = idx_v[pl.ds(k * L, L)] + rebase

    def idx_slice(g):
        return idx_v.at[pl.ds(g * CHUNK, CHUNK)]

    def out_slice(g):
        return out_hbm.at[pl.ds(base + g * CHUNK, CHUNK)]

    pltpu.async_copy(rep_hbm.at[idx_slice(0)], bufs.at[0], gsem)
    pltpu.async_copy(rep_hbm.at[idx_slice(1)], bufs.at[1], gsem)

    for g in range(NCHUNK):
        cur = g % 3
        pltpu.make_async_copy(rep_hbm.at[idx_slice(g)], bufs.at[cur], gsem).wait()
        if g >= 1:
            pltpu.make_async_copy(bufs.at[(g - 1) % 3], out_slice(g - 1), ssem).wait()
        if g + 2 < NCHUNK:
            pltpu.async_copy(rep_hbm.at[idx_slice(g + 2)], bufs.at[(g + 2) % 3], gsem)
        pltpu.async_copy(bufs.at[cur], out_slice(g), ssem)

    pltpu.make_async_copy(bufs.at[(NCHUNK - 1) % 3], out_slice(NCHUNK - 1), ssem).wait()


def kernel(b, cat_ids):
    cat_ids = cat_ids.astype(jnp.int32)
    rep = jnp.tile(b, (2 * NW, 1))
    mesh = plsc.VectorSubcoreMesh(core_axis_name="c", subcore_axis_name="s")
    run = functools.partial(
        pl.kernel,
        mesh=mesh,
        compiler_params=pltpu.CompilerParams(needs_layout_passes=False),
        out_type=jax.ShapeDtypeStruct((B, D), jnp.float32),
        scratch_types=[
            pltpu.VMEM((B_PER_W,), jnp.int32),
            pltpu.VMEM((3, CHUNK, D), jnp.float32),
            pltpu.SemaphoreType.DMA,
            pltpu.SemaphoreType.DMA,
        ],
    )(_gather_body)
    return run(rep, cat_ids)


# speedup vs baseline: 1.0238x; 1.0219x over previous
import functools

import jax
import jax.numpy as jnp
from jax import lax
from jax.experimental import pallas as pl
from jax.experimental.pallas import tpu as pltpu
from jax.experimental.pallas import tpu_sc as plsc

B = 16384
D = 1536
V = 32
L = 16
NC = 2
NS = 16
NW = NC * NS
B_PER_W = B // NW
CHUNK = 16
NCHUNK = B_PER_W // CHUNK


def _gather_body(rep_hbm, idx_hbm, out_hbm, idx_v, bufs, gsem, ssem):
    sid = lax.axis_index("s")
    wid = sid * NC + lax.axis_index("c")
    base = wid * B_PER_W

    pltpu.sync_copy(idx_hbm.at[pl.ds(base, B_PER_W)], idx_v)
    rebase = jnp.broadcast_to(wid * V, (L,)).astype(jnp.int32)
    for k in range(B_PER_W // L):
        idx_v[pl.ds(k * L, L)] = idx_v[pl.ds(k * L, L)] + rebase

    def idx_slice(g):
        return idx_v.at[pl.ds(g * CHUNK, CHUNK)]

    def out_slice(g):
        return out_hbm.at[pl.ds(base + g * CHUNK, CHUNK)]

    pltpu.async_copy(rep_hbm.at[idx_slice(0)], bufs.at[0], gsem)
    pltpu.async_copy(rep_hbm.at[idx_slice(1)], bufs.at[1], gsem)

    for g in range(NCHUNK):
        cur = g % 3
        pltpu.make_async_copy(rep_hbm.at[idx_slice(g)], bufs.at[cur], gsem).wait()
        if g >= 1:
            pltpu.make_async_copy(bufs.at[(g - 1) % 3], out_slice(g - 1), ssem).wait()
        if g + 2 < NCHUNK:
            pltpu.async_copy(rep_hbm.at[idx_slice(g + 2)], bufs.at[(g + 2) % 3], gsem)
        pltpu.async_copy(bufs.at[cur], out_slice(g), ssem)

    pltpu.make_async_copy(bufs.at[(NCHUNK - 1) % 3], out_slice(NCHUNK - 1), ssem).wait()


def kernel(b, cat_ids):
    cat_ids = cat_ids.astype(jnp.int32)
    rep = jnp.tile(b, (NW, 1))
    mesh = plsc.VectorSubcoreMesh(core_axis_name="c", subcore_axis_name="s")
    run = functools.partial(
        pl.kernel,
        mesh=mesh,
        compiler_params=pltpu.CompilerParams(needs_layout_passes=False),
        out_type=jax.ShapeDtypeStruct((B, D), jnp.float32),
        scratch_types=[
            pltpu.VMEM((B_PER_W,), jnp.int32),
            pltpu.VMEM((3, CHUNK, D), jnp.float32),
            pltpu.SemaphoreType.DMA,
            pltpu.SemaphoreType.DMA,
        ],
    )(_gather_body)
    return run(rep, cat_ids)
